# baseline (speedup 1.0000x reference)
import functools

import jax
import jax.numpy as jnp
from jax import lax
from jax.experimental import pallas as pl
from jax.experimental.pallas import tpu as pltpu
from jax.experimental.pallas import tpu_sc as plsc

_B, _H, _W = 16, 384, 384
_POS_WEIGHT = 2.0
_TOTAL = _B * _H * _W
_NC, _NS, _L = 2, 16, 16
_NW = _NC * _NS
_PER_TILE = _TOTAL // _NW
_CHUNK = 9216
_NCHUNK = _PER_TILE // _CHUNK

_C0 = 0.99998187
_C1 = -0.49918785
_C2 = 0.32441181
_C3 = -0.20866966
_C4 = 0.10028721
_C5 = -0.02368925

_mesh = plsc.VectorSubcoreMesh(core_axis_name="c", subcore_axis_name="s")


@functools.partial(
    pl.kernel,
    mesh=_mesh,
    out_type=jax.ShapeDtypeStruct((2, _NW, _L), jnp.float32),
    scratch_types=[
        pltpu.VMEM((_CHUNK,), jnp.float32),
        pltpu.VMEM((_CHUNK,), jnp.float32),
        pltpu.VMEM((_CHUNK,), jnp.float32),
        pltpu.VMEM((_L,), jnp.float32),
        pltpu.VMEM((_L,), jnp.float32),
        pltpu.VMEM((_L,), jnp.float32),
    ],
)
def _partial_loss(x_hbm, p_hbm, n_hbm, sw_hbm, out_hbm, xb, pb, nb, sv, wv, av):
    wid = lax.axis_index("c") * _NS + lax.axis_index("s")
    base = wid * _PER_TILE
    pltpu.sync_copy(sw_hbm.at[0, wid], sv)
    pltpu.sync_copy(sw_hbm.at[1, wid], wv)
    sgn = sv[...]

    def chunk_body(ci, carry):
        acc, cnt = carry
        off = base + ci * _CHUNK
        pltpu.sync_copy(x_hbm.at[pl.ds(off, _CHUNK)], xb)
        pltpu.sync_copy(p_hbm.at[pl.ds(off, _CHUNK)], pb)
        pltpu.sync_copy(n_hbm.at[pl.ds(off, _CHUNK)], nb)

        def inner(i, c2):
            acc2, cnt2 = c2
            sl = pl.ds(i * _L, _L)
            x = xb[sl]
            p = pb[sl]
            n = nb[sl]
            t = x * sgn
            a = jnp.abs(t)
            r = jnp.maximum(t, 0.0)
            u = jnp.exp(-a)
            poly = _C5
            for c in (_C4, _C3, _C2, _C1, _C0):
                poly = poly * u + c
            sp = r + poly * u
            m = jnp.minimum(p, n) > 0.5
            acc2 = acc2 + jnp.where(m, sp, 0.0)
            cnt2 = cnt2 + jnp.where(m, 1.0, 0.0)
            return acc2, cnt2

        return lax.fori_loop(0, _CHUNK // _L, inner, (acc, cnt))

    zero = jnp.zeros((_L,), jnp.float32)
    acc, cnt = lax.fori_loop(0, _NCHUNK, chunk_body, (zero, zero))
    av[...] = acc * wv[...]
    pltpu.sync_copy(av, out_hbm.at[0, wid])
    av[...] = cnt
    pltpu.sync_copy(av, out_hbm.at[1, wid])


@functools.partial(
    pl.kernel,
    mesh=_mesh,
    out_type=jax.ShapeDtypeStruct((_L,), jnp.float32),
    scratch_types=[
        pltpu.VMEM((2, _NW, _L), jnp.float32),
        pltpu.VMEM((_L,), jnp.float32),
    ],
)
def _combine(parts_hbm, out_hbm, buf, outv):
    wid = lax.axis_index("c") * _NS + lax.axis_index("s")

    @pl.when(wid == 0)
    def _():
        pltpu.sync_copy(parts_hbm, buf)
        zero = jnp.zeros((_L,), jnp.float32)

        def body(j, carry):
            a, q = carry
            return a + buf[0, j], q + buf[1, j]

        a, q = lax.fori_loop(0, _NW, body, (zero, zero))
        sa = a[0]
        sq = q[0]
        for i in range(1, _L):
            sa = sa + a[i]
            sq = sq + q[i]
        outv[...] = jnp.broadcast_to(sa, (_L,)) / jnp.broadcast_to(sq, (_L,))
        pltpu.sync_copy(outv, out_hbm)


def kernel(cancer_logits, prostate_mask, needle_mask, label, involvement):
    del involvement
    x = cancer_logits.reshape(_TOTAL)
    p = prostate_mask.reshape(_TOTAL)
    n = needle_mask.reshape(_TOTAL)
    is_pos = (label == 1)
    sign = jnp.where(is_pos, -1.0, 1.0).astype(jnp.float32)
    wgt = jnp.where(is_pos, _POS_WEIGHT, 1.0).astype(jnp.float32)
    sw = jnp.stack([
        jnp.broadcast_to(jnp.repeat(sign, _NW // _B)[:, None], (_NW, _L)),
        jnp.broadcast_to(jnp.repeat(wgt, _NW // _B)[:, None], (_NW, _L)),
    ])
    parts = _partial_loss(x, p, n, sw)
    return _combine(parts)[0]


# speedup vs baseline: 1.3823x; 1.3823x over previous
import functools

import jax
import jax.numpy as jnp
from jax import lax
from jax.experimental import pallas as pl
from jax.experimental.pallas import tpu as pltpu
from jax.experimental.pallas import tpu_sc as plsc

_B, _H, _W = 16, 384, 384
_POS_WEIGHT = 2.0
_ROWS = _B * _H
_NC, _NS, _L = 2, 16, 16
_NW = _NC * _NS
_TROWS = _ROWS // _NW
_CROWS = 24
_NCHUNK = _TROWS // _CROWS
_CVECS = _CROWS * _W // _L

_C0 = 0.99998187
_C1 = -0.49918785
_C2 = 0.32441181
_C3 = -0.20866966
_C4 = 0.10028721
_C5 = -0.02368925

_mesh = plsc.VectorSubcoreMesh(core_axis_name="c", subcore_axis_name="s")


@functools.partial(
    pl.kernel,
    mesh=_mesh,
    out_type=jax.ShapeDtypeStruct((2, _NW, _L), jnp.float32),
    scratch_types=[
        pltpu.VMEM((_CROWS, _W), jnp.float32),
        pltpu.VMEM((_CROWS, _W), jnp.float32),
        pltpu.VMEM((_CROWS, _W), jnp.float32),
        pltpu.VMEM((_L,), jnp.float32),
        pltpu.VMEM((_L,), jnp.float32),
        pltpu.VMEM((_L,), jnp.float32),
    ],
)
def _partial_loss(x_hbm, p_hbm, n_hbm, sw_hbm, out_hbm, xb, pb, nb, sv, wv, av):
    wid = lax.axis_index("c") * _NS + lax.axis_index("s")
    base = wid * _TROWS
    pltpu.sync_copy(sw_hbm.at[0, wid], sv)
    pltpu.sync_copy(sw_hbm.at[1, wid], wv)
    sgn = sv[...]

    def chunk_body(ci, carry):
        acc, cnt = carry
        off = base + ci * _CROWS
        pltpu.sync_copy(x_hbm.at[pl.ds(off, _CROWS), :], xb)
        pltpu.sync_copy(p_hbm.at[pl.ds(off, _CROWS), :], pb)
        pltpu.sync_copy(n_hbm.at[pl.ds(off, _CROWS), :], nb)

        def inner(i, c2):
            acc2, cnt2 = c2
            r = i // (_W // _L)
            sl = pl.ds((i % (_W // _L)) * _L, _L)
            x = xb[r, sl]
            p = pb[r, sl]
            n = nb[r, sl]
            t = x * sgn
            a = jnp.abs(t)
            r = jnp.maximum(t, 0.0)
            u = jnp.exp(-a)
            poly = _C5
            for c in (_C4, _C3, _C2, _C1, _C0):
                poly = poly * u + c
            sp = r + poly * u
            m = jnp.minimum(p, n) > 0.5
            acc2 = acc2 + jnp.where(m, sp, 0.0)
            cnt2 = cnt2 + jnp.where(m, 1.0, 0.0)
            return acc2, cnt2

        return lax.fori_loop(0, _CVECS, inner, (acc, cnt))

    zero = jnp.zeros((_L,), jnp.float32)
    acc, cnt = lax.fori_loop(0, _NCHUNK, chunk_body, (zero, zero))
    av[...] = acc * wv[...]
    pltpu.sync_copy(av, out_hbm.at[0, wid])
    av[...] = cnt
    pltpu.sync_copy(av, out_hbm.at[1, wid])


@functools.partial(
    pl.kernel,
    mesh=_mesh,
    out_type=jax.ShapeDtypeStruct((_L,), jnp.float32),
    scratch_types=[
        pltpu.VMEM((2, _NW, _L), jnp.float32),
        pltpu.VMEM((_L,), jnp.float32),
    ],
)
def _combine(parts_hbm, out_hbm, buf, outv):
    wid = lax.axis_index("c") * _NS + lax.axis_index("s")

    @pl.when(wid == 0)
    def _():
        pltpu.sync_copy(parts_hbm, buf)
        zero = jnp.zeros((_L,), jnp.float32)

        def body(j, carry):
            a, q = carry
            return a + buf[0, j], q + buf[1, j]

        a, q = lax.fori_loop(0, _NW, body, (zero, zero))
        sa = a[0]
        sq = q[0]
        for i in range(1, _L):
            sa = sa + a[i]
            sq = sq + q[i]
        outv[...] = jnp.broadcast_to(sa, (_L,)) / jnp.broadcast_to(sq, (_L,))
        pltpu.sync_copy(outv, out_hbm)


def kernel(cancer_logits, prostate_mask, needle_mask, label, involvement):
    del involvement
    x = cancer_logits.reshape(_ROWS, _W)
    p = prostate_mask.reshape(_ROWS, _W)
    n = needle_mask.reshape(_ROWS, _W)
    is_pos = (label == 1)
    sign = jnp.where(is_pos, -1.0, 1.0).astype(jnp.float32)
    wgt = jnp.where(is_pos, _POS_WEIGHT, 1.0).astype(jnp.float32)
    sw = jnp.stack([
        jnp.broadcast_to(jnp.repeat(sign, _NW // _B)[:, None], (_NW, _L)),
        jnp.broadcast_to(jnp.repeat(wgt, _NW // _B)[:, None], (_NW, _L)),
    ])
    parts = _partial_loss(x, p, n, sw)
    return _combine(parts)[0]


# speedup vs baseline: 1.9554x; 1.4146x over previous
import functools

import jax
import jax.numpy as jnp
from jax import lax
from jax.experimental import pallas as pl
from jax.experimental.pallas import tpu as pltpu
from jax.experimental.pallas import tpu_sc as plsc

_B, _H, _W = 16, 384, 384
_POS_WEIGHT = 2.0
_ROWS = _B * _H
_NC, _NS, _L = 2, 16, 16
_NW = _NC * _NS
_TROWS = _ROWS // _NW
_CROWS = 48
_NCHUNK = _TROWS // _CROWS
_CVECS = _CROWS * _W // _L

_C0 = 0.99988787
_C1 = -0.49636774
_C2 = 0.30467086
_C3 = -0.15602694
_C4 = 0.04106407

_mesh = plsc.VectorSubcoreMesh(core_axis_name="c", subcore_axis_name="s")


@functools.partial(
    pl.kernel,
    mesh=_mesh,
    out_type=jax.ShapeDtypeStruct((2, _NW, _L), jnp.float32),
    scratch_types=[
        pltpu.VMEM((2, _CROWS, _W), jnp.float32),
        pltpu.VMEM((2, _CROWS, _W), jnp.float32),
        pltpu.VMEM((2, _CROWS, _W), jnp.float32),
        pltpu.VMEM((_L,), jnp.float32),
        pltpu.VMEM((_L,), jnp.float32),
        pltpu.VMEM((_L,), jnp.float32),
        pltpu.SemaphoreType.DMA,
        pltpu.SemaphoreType.DMA,
        pltpu.SemaphoreType.DMA,
        pltpu.SemaphoreType.DMA,
        pltpu.SemaphoreType.DMA,
        pltpu.SemaphoreType.DMA,
    ],
)
def _partial_loss(x_hbm, p_hbm, n_hbm, sw_hbm, out_hbm, xb, pb, nb, sv, wv, av,
                  sx0, sp0, sn0, sx1, sp1, sn1):
    wid = lax.axis_index("c") * _NS + lax.axis_index("s")
    base = wid * _TROWS
    pltpu.sync_copy(sw_hbm.at[0, wid], sv)
    pltpu.sync_copy(sw_hbm.at[1, wid], wv)
    sgn = sv[...]
    sems = ((sx0, sp0, sn0), (sx1, sp1, sn1))

    def issue(ci, b):
        off = base + ci * _CROWS
        sl = pl.ds(off, _CROWS)
        return (
            pltpu.async_copy(x_hbm.at[sl, :], xb.at[b], sems[b][0]),
            pltpu.async_copy(p_hbm.at[sl, :], pb.at[b], sems[b][1]),
            pltpu.async_copy(n_hbm.at[sl, :], nb.at[b], sems[b][2]),
        )

    def compute(b, carry):
        def inner(i, c2):
            acc2, cnt2 = c2
            r = i // (_W // _L)
            sl = pl.ds((i % (_W // _L)) * _L, _L)
            x = xb[b, r, sl]
            p = pb[b, r, sl]
            n = nb[b, r, sl]
            t = x * sgn
            a = jnp.abs(t)
            rl = jnp.maximum(t, 0.0)
            u = jnp.exp(-a)
            poly = _C4
            for c in (_C3, _C2, _C1, _C0):
                poly = poly * u + c
            sp = rl + poly * u
            m = jnp.minimum(p, n) > 0.5
            acc2 = acc2 + jnp.where(m, sp, 0.0)
            cnt2 = cnt2 + jnp.where(m, 1.0, 0.0)
            return acc2, cnt2

        return lax.fori_loop(0, _CVECS, inner, carry)

    zero = jnp.zeros((_L,), jnp.float32)
    acc, cnt = zero, zero
    pend = issue(0, 0)
    for ci in range(_NCHUNK):
        b = ci % 2
        nxt = issue(ci + 1, 1 - b) if ci + 1 < _NCHUNK else None
        for h in pend:
            h.wait()
        acc, cnt = compute(b, (acc, cnt))
        pend = nxt
    av[...] = acc * wv[...]
    pltpu.sync_copy(av, out_hbm.at[0, wid])
    av[...] = cnt
    pltpu.sync_copy(av, out_hbm.at[1, wid])


@functools.partial(
    pl.kernel,
    mesh=_mesh,
    out_type=jax.ShapeDtypeStruct((_L,), jnp.float32),
    scratch_types=[
        pltpu.VMEM((2, _NW, _L), jnp.float32),
        pltpu.VMEM((_L,), jnp.float32),
    ],
)
def _combine(parts_hbm, out_hbm, buf, outv):
    wid = lax.axis_index("c") * _NS + lax.axis_index("s")

    @pl.when(wid == 0)
    def _():
        pltpu.sync_copy(parts_hbm, buf)
        zero = jnp.zeros((_L,), jnp.float32)

        def body(j, carry):
            a, q = carry
            return a + buf[0, j], q + buf[1, j]

        a, q = lax.fori_loop(0, _NW, body, (zero, zero))
        sa = a[0]
        sq = q[0]
        for i in range(1, _L):
            sa = sa + a[i]
            sq = sq + q[i]
        outv[...] = jnp.broadcast_to(sa, (_L,)) / jnp.broadcast_to(sq, (_L,))
        pltpu.sync_copy(outv, out_hbm)


def kernel(cancer_logits, prostate_mask, needle_mask, label, involvement):
    del involvement
    x = cancer_logits.reshape(_ROWS, _W)
    p = prostate_mask.reshape(_ROWS, _W)
    n = needle_mask.reshape(_ROWS, _W)
    is_pos = (label == 1)
    sign = jnp.where(is_pos, -1.0, 1.0).astype(jnp.float32)
    wgt = jnp.where(is_pos, _POS_WEIGHT, 1.0).astype(jnp.float32)
    sw = jnp.stack([
        jnp.broadcast_to(jnp.repeat(sign, _NW // _B)[:, None], (_NW, _L)),
        jnp.broadcast_to(jnp.repeat(wgt, _NW // _B)[:, None], (_NW, _L)),
    ])
    parts = _partial_loss(x, p, n, sw)
    return _combine(parts)[0]


# speedup vs baseline: 2.3681x; 1.2111x over previous
import functools

import jax
import jax.numpy as jnp
from jax import lax
from jax.experimental import pallas as pl
from jax.experimental.pallas import tpu as pltpu
from jax.experimental.pallas import tpu_sc as plsc

_B, _H, _W = 16, 384, 384
_POS_WEIGHT = 2.0
_ROWS = _B * _H
_NC, _NS, _L = 2, 16, 16
_NW = _NC * _NS

_K = 4
_TC_IMGS = _B - _K
_TC_ROWS = _TC_IMGS * _H
_TROWS = _K * _H // _NW
_CROWS = _TROWS if _TROWS < 48 else 24
_NCHUNK = _TROWS // _CROWS
_CVECS = _CROWS * _W // _L

_BR = 192
_TC_GRID = _TC_ROWS // _BR

_C0 = 0.99988787
_C1 = -0.49636774
_C2 = 0.30467086
_C3 = -0.15602694
_C4 = 0.04106407

_mesh = plsc.VectorSubcoreMesh(core_axis_name="c", subcore_axis_name="s")


@functools.partial(
    pl.kernel,
    mesh=_mesh,
    out_type=jax.ShapeDtypeStruct((2, _NW, _L), jnp.float32),
    scratch_types=[
        pltpu.VMEM((2, _CROWS, _W), jnp.float32),
        pltpu.VMEM((2, _CROWS, _W), jnp.float32),
        pltpu.VMEM((2, _CROWS, _W), jnp.float32),
        pltpu.VMEM((_L,), jnp.float32),
        pltpu.VMEM((_L,), jnp.float32),
        pltpu.VMEM((_L,), jnp.float32),
        pltpu.SemaphoreType.DMA,
        pltpu.SemaphoreType.DMA,
        pltpu.SemaphoreType.DMA,
        pltpu.SemaphoreType.DMA,
        pltpu.SemaphoreType.DMA,
        pltpu.SemaphoreType.DMA,
    ],
)
def _sc_partial(x_hbm, p_hbm, n_hbm, sw_hbm, out_hbm, xb, pb, nb, sv, wv, av,
                sx0, sp0, sn0, sx1, sp1, sn1):
    wid = lax.axis_index("c") * _NS + lax.axis_index("s")
    base = _TC_ROWS + wid * _TROWS
    pltpu.sync_copy(sw_hbm.at[0, wid], sv)
    pltpu.sync_copy(sw_hbm.at[1, wid], wv)
    sgn = sv[...]
    sems = ((sx0, sp0, sn0), (sx1, sp1, sn1))

    def issue(ci, b):
        sl = pl.ds(base + ci * _CROWS, _CROWS)
        return (
            pltpu.async_copy(x_hbm.at[sl, :], xb.at[b], sems[b][0]),
            pltpu.async_copy(p_hbm.at[sl, :], pb.at[b], sems[b][1]),
            pltpu.async_copy(n_hbm.at[sl, :], nb.at[b], sems[b][2]),
        )

    def compute(b, carry):
        def inner(i, c2):
            acc2, cnt2 = c2
            r = i // (_W // _L)
            sl = pl.ds((i % (_W // _L)) * _L, _L)
            x = xb[b, r, sl]
            p = pb[b, r, sl]
            n = nb[b, r, sl]
            t = x * sgn
            a = jnp.abs(t)
            rl = jnp.maximum(t, 0.0)
            u = jnp.exp(-a)
            poly = _C4
            for c in (_C3, _C2, _C1, _C0):
                poly = poly * u + c
            sp = rl + poly * u
            m = jnp.minimum(p, n) > 0.5
            acc2 = acc2 + jnp.where(m, sp, 0.0)
            cnt2 = cnt2 + jnp.where(m, 1.0, 0.0)
            return acc2, cnt2

        return lax.fori_loop(0, _CVECS, inner, carry)

    zero = jnp.zeros((_L,), jnp.float32)
    acc, cnt = zero, zero
    pend = issue(0, 0)
    for ci in range(_NCHUNK):
        b = ci % 2
        nxt = issue(ci + 1, 1 - b) if ci + 1 < _NCHUNK else None
        for h in pend:
            h.wait()
        acc, cnt = compute(b, (acc, cnt))
        pend = nxt
    av[...] = acc * wv[...]
    pltpu.sync_copy(av, out_hbm.at[0, wid])
    av[...] = cnt
    pltpu.sync_copy(av, out_hbm.at[1, wid])


def _tc_body(sgn_ref, wgt_ref, x_ref, p_ref, n_ref, os_ref, oc_ref):
    i = pl.program_id(0)
    img = (i * _BR) // _H
    sgn = sgn_ref[img]
    wgt = wgt_ref[img]
    x = x_ref[...]
    t = x * sgn
    sp = jnp.maximum(t, 0.0) + jnp.log1p(jnp.exp(-jnp.abs(t)))
    m = (p_ref[...] > 0.5) & (n_ref[...] > 0.5)
    s_blk = jnp.sum(jnp.where(m, sp, 0.0)) * wgt
    c_blk = jnp.sum(jnp.where(m, 1.0, 0.0))

    @pl.when(i == 0)
    def _():
        os_ref[0, 0] = 0.0
        oc_ref[0, 0] = 0.0

    os_ref[0, 0] += s_blk
    oc_ref[0, 0] += c_blk


_tc_partial = pl.pallas_call(
    _tc_body,
    grid_spec=pltpu.PrefetchScalarGridSpec(
        num_scalar_prefetch=2,
        grid=(_TC_GRID,),
        in_specs=[
            pl.BlockSpec((_BR, _W), lambda i, *_: (i, 0)),
            pl.BlockSpec((_BR, _W), lambda i, *_: (i, 0)),
            pl.BlockSpec((_BR, _W), lambda i, *_: (i, 0)),
        ],
        out_specs=[
            pl.BlockSpec(memory_space=pltpu.SMEM),
            pl.BlockSpec(memory_space=pltpu.SMEM),
        ],
    ),
    out_shape=[
        jax.ShapeDtypeStruct((1, 1), jnp.float32),
        jax.ShapeDtypeStruct((1, 1), jnp.float32),
    ],
)


def _combine_body(parts_ref, ts_ref, tc_ref, out_ref):
    ps = parts_ref[...]
    num = jnp.sum(ps[0]) + ts_ref[0, 0]
    den = jnp.sum(ps[1]) + tc_ref[0, 0]
    out_ref[0, 0] = num / den


_combine = pl.pallas_call(
    _combine_body,
    in_specs=[
        pl.BlockSpec((2, _NW, _L), lambda: (0, 0, 0)),
        pl.BlockSpec(memory_space=pltpu.SMEM),
        pl.BlockSpec(memory_space=pltpu.SMEM),
    ],
    out_specs=pl.BlockSpec(memory_space=pltpu.SMEM),
    out_shape=jax.ShapeDtypeStruct((1, 1), jnp.float32),
)


def kernel(cancer_logits, prostate_mask, needle_mask, label, involvement):
    del involvement
    x = cancer_logits.reshape(_ROWS, _W)
    p = prostate_mask.reshape(_ROWS, _W)
    n = needle_mask.reshape(_ROWS, _W)
    is_pos = (label == 1)
    sign = jnp.where(is_pos, -1.0, 1.0).astype(jnp.float32)
    wgt = jnp.where(is_pos, _POS_WEIGHT, 1.0).astype(jnp.float32)
    tile_img = _TC_IMGS + (jnp.arange(_NW) * _TROWS) // _H
    sw = jnp.stack([
        jnp.broadcast_to(sign[tile_img][:, None], (_NW, _L)),
        jnp.broadcast_to(wgt[tile_img][:, None], (_NW, _L)),
    ])
    parts = _sc_partial(x, p, n, sw)
    tc_s, tc_c = _tc_partial(sign, wgt, x, p, n)
    loss = _combine(parts, tc_s, tc_c)
    return loss[0, 0]


# speedup vs baseline: 2.7243x; 1.1504x over previous
import functools

import jax
import jax.numpy as jnp
from jax import lax
from jax.experimental import pallas as pl
from jax.experimental.pallas import tpu as pltpu
from jax.experimental.pallas import tpu_sc as plsc

_B, _H, _W = 16, 384, 384
_POS_WEIGHT = 2.0
_ROWS = _B * _H
_NC, _NS, _L = 2, 16, 16
_NW = _NC * _NS

_K = 4
_TC_IMGS = _B - _K
_TC_ROWS = _TC_IMGS * _H
_TROWS = _K * _H // _NW
_CROWS = _TROWS if _TROWS < 48 else 24
_NCHUNK = _TROWS // _CROWS
_CVECS = _CROWS * _W // _L

_BR = 384
_TC_GRID = _TC_ROWS // _BR

_C0 = 0.99988787
_C1 = -0.49636774
_C2 = 0.30467086
_C3 = -0.15602694
_C4 = 0.04106407

_mesh = plsc.VectorSubcoreMesh(core_axis_name="c", subcore_axis_name="s")


@functools.partial(
    pl.kernel,
    mesh=_mesh,
    out_type=jax.ShapeDtypeStruct((2, _NW, _L), jnp.float32),
    scratch_types=[
        pltpu.VMEM((2, _CROWS, _W), jnp.float32),
        pltpu.VMEM((2, _CROWS, _W), jnp.float32),
        pltpu.VMEM((2, _CROWS, _W), jnp.float32),
        pltpu.VMEM((_L,), jnp.float32),
        pltpu.VMEM((_L,), jnp.float32),
        pltpu.VMEM((_L,), jnp.float32),
        pltpu.SemaphoreType.DMA,
        pltpu.SemaphoreType.DMA,
        pltpu.SemaphoreType.DMA,
        pltpu.SemaphoreType.DMA,
        pltpu.SemaphoreType.DMA,
        pltpu.SemaphoreType.DMA,
    ],
)
def _sc_partial(x_hbm, p_hbm, n_hbm, sw_hbm, out_hbm, xb, pb, nb, sv, wv, av,
                sx0, sp0, sn0, sx1, sp1, sn1):
    wid = lax.axis_index("c") * _NS + lax.axis_index("s")
    base = _TC_ROWS + wid * _TROWS
    pltpu.sync_copy(sw_hbm.at[0, wid], sv)
    pltpu.sync_copy(sw_hbm.at[1, wid], wv)
    sgn = sv[...]
    sems = ((sx0, sp0, sn0), (sx1, sp1, sn1))

    def issue(ci, b):
        sl = pl.ds(base + ci * _CROWS, _CROWS)
        return (
            pltpu.async_copy(x_hbm.at[sl, :], xb.at[b], sems[b][0]),
            pltpu.async_copy(p_hbm.at[sl, :], pb.at[b], sems[b][1]),
            pltpu.async_copy(n_hbm.at[sl, :], nb.at[b], sems[b][2]),
        )

    def compute(b, carry):
        def inner(i, c2):
            acc2, cnt2 = c2
            r = i // (_W // _L)
            sl = pl.ds((i % (_W // _L)) * _L, _L)
            x = xb[b, r, sl]
            p = pb[b, r, sl]
            n = nb[b, r, sl]
            t = x * sgn
            a = jnp.abs(t)
            rl = jnp.maximum(t, 0.0)
            u = jnp.exp(-a)
            poly = _C4
            for c in (_C3, _C2, _C1, _C0):
                poly = poly * u + c
            sp = rl + poly * u
            m = jnp.minimum(p, n) > 0.5
            acc2 = acc2 + jnp.where(m, sp, 0.0)
            cnt2 = cnt2 + jnp.where(m, 1.0, 0.0)
            return acc2, cnt2

        return lax.fori_loop(0, _CVECS, inner, carry)

    zero = jnp.zeros((_L,), jnp.float32)
    acc, cnt = zero, zero
    pend = issue(0, 0)
    for ci in range(_NCHUNK):
        b = ci % 2
        nxt = issue(ci + 1, 1 - b) if ci + 1 < _NCHUNK else None
        for h in pend:
            h.wait()
        acc, cnt = compute(b, (acc, cnt))
        pend = nxt
    av[...] = acc * wv[...]
    pltpu.sync_copy(av, out_hbm.at[0, wid])
    av[...] = cnt
    pltpu.sync_copy(av, out_hbm.at[1, wid])


def _tc_body(sgn_ref, wgt_ref, x_ref, p_ref, n_ref, os_ref, oc_ref):
    i = pl.program_id(0)
    img = (i * _BR) // _H
    sgn = sgn_ref[img]
    wgt = wgt_ref[img]
    x = x_ref[...]
    t = x * sgn
    sp = jnp.maximum(t, 0.0) + jnp.log1p(jnp.exp(-jnp.abs(t)))
    m = (p_ref[...] > 0.5) & (n_ref[...] > 0.5)
    s_blk = jnp.sum(jnp.where(m, sp, 0.0)) * wgt
    c_blk = jnp.sum(jnp.where(m, 1.0, 0.0))

    @pl.when(i == 0)
    def _():
        os_ref[0, 0] = 0.0
        oc_ref[0, 0] = 0.0

    os_ref[0, 0] += s_blk
    oc_ref[0, 0] += c_blk


_tc_partial = pl.pallas_call(
    _tc_body,
    grid_spec=pltpu.PrefetchScalarGridSpec(
        num_scalar_prefetch=2,
        grid=(_TC_GRID,),
        in_specs=[
            pl.BlockSpec((_BR, _W), lambda i, *_: (i, 0)),
            pl.BlockSpec((_BR, _W), lambda i, *_: (i, 0)),
            pl.BlockSpec((_BR, _W), lambda i, *_: (i, 0)),
        ],
        out_specs=[
            pl.BlockSpec(memory_space=pltpu.SMEM),
            pl.BlockSpec(memory_space=pltpu.SMEM),
        ],
    ),
    out_shape=[
        jax.ShapeDtypeStruct((1, 1), jnp.float32),
        jax.ShapeDtypeStruct((1, 1), jnp.float32),
    ],
)


def _combine_body(parts_ref, ts_ref, tc_ref, out_ref):
    ps = parts_ref[...]
    num = jnp.sum(ps[0]) + ts_ref[0, 0]
    den = jnp.sum(ps[1]) + tc_ref[0, 0]
    out_ref[0, 0] = num / den


_combine = pl.pallas_call(
    _combine_body,
    in_specs=[
        pl.BlockSpec((2, _NW, _L), lambda: (0, 0, 0)),
        pl.BlockSpec(memory_space=pltpu.SMEM),
        pl.BlockSpec(memory_space=pltpu.SMEM),
    ],
    out_specs=pl.BlockSpec(memory_space=pltpu.SMEM),
    out_shape=jax.ShapeDtypeStruct((1, 1), jnp.float32),
)


def kernel(cancer_logits, prostate_mask, needle_mask, label, involvement):
    del involvement
    x = cancer_logits.reshape(_ROWS, _W)
    p = prostate_mask.reshape(_ROWS, _W)
    n = needle_mask.reshape(_ROWS, _W)
    is_pos = (label == 1)
    sign = jnp.where(is_pos, -1.0, 1.0).astype(jnp.float32)
    wgt = jnp.where(is_pos, _POS_WEIGHT, 1.0).astype(jnp.float32)
    tile_img = _TC_IMGS + (jnp.arange(_NW) * _TROWS) // _H
    sw = jnp.stack([
        jnp.broadcast_to(sign[tile_img][:, None], (_NW, _L)),
        jnp.broadcast_to(wgt[tile_img][:, None], (_NW, _L)),
    ])
    parts = _sc_partial(x, p, n, sw)
    tc_s, tc_c = _tc_partial(sign, wgt, x, p, n)
    loss = _combine(parts, tc_s, tc_c)
    return loss[0, 0]


# speedup vs baseline: 2.8119x; 1.0322x over previous
import functools

import jax
import jax.numpy as jnp
from jax import lax
from jax.experimental import pallas as pl
from jax.experimental.pallas import tpu as pltpu
from jax.experimental.pallas import tpu_sc as plsc

_B, _H, _W = 16, 384, 384
_POS_WEIGHT = 2.0
_ROWS = _B * _H
_NC, _NS, _L = 1, 16, 16
_NW = _NC * _NS

_K = 4
_TC_IMGS = _B - _K
_TC_ROWS = _TC_IMGS * _H
_TROWS = _K * _H // _NW
_CROWS = _TROWS if _TROWS < 48 else 24
_NCHUNK = _TROWS // _CROWS
_CVECS = _CROWS * _W // _L

_BR = 384
_TC_GRID = _TC_ROWS // _BR

_C0 = 0.99988787
_C1 = -0.49636774
_C2 = 0.30467086
_C3 = -0.15602694
_C4 = 0.04106407

_mesh = plsc.VectorSubcoreMesh(
    core_axis_name="c", subcore_axis_name="s", num_cores=_NC)


@functools.partial(
    pl.kernel,
    mesh=_mesh,
    out_type=jax.ShapeDtypeStruct((2, _NW, _L), jnp.float32),
    scratch_types=[
        pltpu.VMEM((2, _CROWS, _W), jnp.float32),
        pltpu.VMEM((2, _CROWS, _W), jnp.float32),
        pltpu.VMEM((2, _CROWS, _W), jnp.float32),
        pltpu.VMEM((_L,), jnp.float32),
        pltpu.VMEM((_L,), jnp.float32),
        pltpu.VMEM((_L,), jnp.float32),
        pltpu.SemaphoreType.DMA,
        pltpu.SemaphoreType.DMA,
        pltpu.SemaphoreType.DMA,
        pltpu.SemaphoreType.DMA,
        pltpu.SemaphoreType.DMA,
        pltpu.SemaphoreType.DMA,
    ],
)
def _sc_partial(x_hbm, p_hbm, n_hbm, sw_hbm, out_hbm, xb, pb, nb, sv, wv, av,
                sx0, sp0, sn0, sx1, sp1, sn1):
    wid = lax.axis_index("c") * _NS + lax.axis_index("s")
    base = _TC_ROWS + wid * _TROWS
    pltpu.sync_copy(sw_hbm.at[0, wid], sv)
    pltpu.sync_copy(sw_hbm.at[1, wid], wv)
    sgn = sv[...]
    sems = ((sx0, sp0, sn0), (sx1, sp1, sn1))

    def issue(ci, b):
        sl = pl.ds(base + ci * _CROWS, _CROWS)
        return (
            pltpu.async_copy(x_hbm.at[sl, :], xb.at[b], sems[b][0]),
            pltpu.async_copy(p_hbm.at[sl, :], pb.at[b], sems[b][1]),
            pltpu.async_copy(n_hbm.at[sl, :], nb.at[b], sems[b][2]),
        )

    def compute(b, carry):
        def inner(i, c2):
            acc2, cnt2 = c2
            r = i // (_W // _L)
            sl = pl.ds((i % (_W // _L)) * _L, _L)
            x = xb[b, r, sl]
            p = pb[b, r, sl]
            n = nb[b, r, sl]
            t = x * sgn
            a = jnp.abs(t)
            rl = jnp.maximum(t, 0.0)
            u = jnp.exp(-a)
            poly = _C4
            for c in (_C3, _C2, _C1, _C0):
                poly = poly * u + c
            sp = rl + poly * u
            m = jnp.minimum(p, n) > 0.5
            acc2 = acc2 + jnp.where(m, sp, 0.0)
            cnt2 = cnt2 + jnp.where(m, 1.0, 0.0)
            return acc2, cnt2

        return lax.fori_loop(0, _CVECS, inner, carry)

    zero = jnp.zeros((_L,), jnp.float32)
    acc, cnt = zero, zero
    pend = issue(0, 0)
    for ci in range(_NCHUNK):
        b = ci % 2
        nxt = issue(ci + 1, 1 - b) if ci + 1 < _NCHUNK else None
        for h in pend:
            h.wait()
        acc, cnt = compute(b, (acc, cnt))
        pend = nxt
    av[...] = acc * wv[...]
    pltpu.sync_copy(av, out_hbm.at[0, wid])
    av[...] = cnt
    pltpu.sync_copy(av, out_hbm.at[1, wid])


def _tc_body(sgn_ref, wgt_ref, x_ref, p_ref, n_ref, os_ref, oc_ref):
    i = pl.program_id(0)
    img = (i * _BR) // _H
    sgn = sgn_ref[img]
    wgt = wgt_ref[img]
    x = x_ref[...]
    t = x * sgn
    sp = jnp.maximum(t, 0.0) + jnp.log1p(jnp.exp(-jnp.abs(t)))
    m = (p_ref[...] > 0.5) & (n_ref[...] > 0.5)
    s_blk = jnp.sum(jnp.where(m, sp, 0.0)) * wgt
    c_blk = jnp.sum(jnp.where(m, 1.0, 0.0))

    @pl.when(i == 0)
    def _():
        os_ref[0, 0] = 0.0
        oc_ref[0, 0] = 0.0

    os_ref[0, 0] += s_blk
    oc_ref[0, 0] += c_blk


_tc_partial = pl.pallas_call(
    _tc_body,
    grid_spec=pltpu.PrefetchScalarGridSpec(
        num_scalar_prefetch=2,
        grid=(_TC_GRID,),
        in_specs=[
            pl.BlockSpec((_BR, _W), lambda i, *_: (i, 0)),
            pl.BlockSpec((_BR, _W), lambda i, *_: (i, 0)),
            pl.BlockSpec((_BR, _W), lambda i, *_: (i, 0)),
        ],
        out_specs=[
            pl.BlockSpec(memory_space=pltpu.SMEM),
            pl.BlockSpec(memory_space=pltpu.SMEM),
        ],
    ),
    out_shape=[
        jax.ShapeDtypeStruct((1, 1), jnp.float32),
        jax.ShapeDtypeStruct((1, 1), jnp.float32),
    ],
)


def _combine_body(parts_ref, ts_ref, tc_ref, out_ref):
    ps = parts_ref[...]
    num = jnp.sum(ps[0]) + ts_ref[0, 0]
    den = jnp.sum(ps[1]) + tc_ref[0, 0]
    out_ref[0, 0] = num / den


_combine = pl.pallas_call(
    _combine_body,
    in_specs=[
        pl.BlockSpec((2, _NW, _L), lambda: (0, 0, 0)),
        pl.BlockSpec(memory_space=pltpu.SMEM),
        pl.BlockSpec(memory_space=pltpu.SMEM),
    ],
    out_specs=pl.BlockSpec(memory_space=pltpu.SMEM),
    out_shape=jax.ShapeDtypeStruct((1, 1), jnp.float32),
)


def kernel(cancer_logits, prostate_mask, needle_mask, label, involvement):
    del involvement
    x = cancer_logits.reshape(_ROWS, _W)
    p = prostate_mask.reshape(_ROWS, _W)
    n = needle_mask.reshape(_ROWS, _W)
    is_pos = (label == 1)
    sign = jnp.where(is_pos, -1.0, 1.0).astype(jnp.float32)
    wgt = jnp.where(is_pos, _POS_WEIGHT, 1.0).astype(jnp.float32)
    tile_img = _TC_IMGS + (jnp.arange(_NW) * _TROWS) // _H
    sw = jnp.stack([
        jnp.broadcast_to(sign[tile_img][:, None], (_NW, _L)),
        jnp.broadcast_to(wgt[tile_img][:, None], (_NW, _L)),
    ])
    parts = _sc_partial(x, p, n, sw)
    tc_s, tc_c = _tc_partial(sign, wgt, x, p, n)
    loss = _combine(parts, tc_s, tc_c)
    return loss[0, 0]
